# baseline (speedup 1.0000x reference)
import functools

import jax
import jax.numpy as jnp
from jax import lax
from jax.experimental import pallas as pl
from jax.experimental.pallas import tpu as pltpu
from jax.experimental.pallas import tpu_sc as plsc

_B, _L, _EMB = 4096, 200, 100
_DP = 128
_NC, _NS = 2, 16
_NW = _NC * _NS
_BTOT = _B * _L
_BPW = _BTOT // _NW
_C = 128
_NCHUNK = _BPW // _C
_NBUF = 4

_mesh = plsc.VectorSubcoreMesh(core_axis_name="c", subcore_axis_name="s")


@functools.partial(
    pl.kernel,
    out_type=jax.ShapeDtypeStruct((_BTOT, _DP), jnp.float32),
    mesh=_mesh,
    scratch_types=[
        pltpu.VMEM((_NCHUNK, _C), jnp.int32),
        *[pltpu.VMEM((_C, _DP), jnp.float32) for _ in range(_NBUF)],
        *[pltpu.SemaphoreType.DMA for _ in range(2 * _NBUF)],
    ],
    compiler_params=pltpu.CompilerParams(use_tc_tiling_on_sc=True),
)
def _gather(idx_hbm, table_hbm, out_hbm, idx_v, *bufs_and_sems):
    rows = bufs_and_sems[:_NBUF]
    gsems = bufs_and_sems[_NBUF:2 * _NBUF]
    osems = bufs_and_sems[2 * _NBUF:]
    wid = lax.axis_index("s") * _NC + lax.axis_index("c")
    pltpu.sync_copy(idx_hbm.at[pl.ds(wid * _NCHUNK, _NCHUNK)], idx_v)
    base = wid * _BPW

    for b in range(_NBUF):
        pltpu.async_copy(table_hbm.at[idx_v.at[b]], rows[b], gsems[b])

    def group(go, carry):
        for b in range(_NBUF):
            g = go * _NBUF + b
            pltpu.make_async_copy(
                table_hbm.at[idx_v.at[0]], rows[b], gsems[b]).wait()
            pltpu.async_copy(
                rows[b], out_hbm.at[pl.ds((base + g * _C), _C)], osems[b])

            @pl.when(go < _NCHUNK // _NBUF - 1)
            def _():
                pltpu.make_async_copy(
                    rows[b], out_hbm.at[pl.ds(0, _C)], osems[b]).wait()
                pltpu.async_copy(
                    table_hbm.at[idx_v.at[g + _NBUF]], rows[b], gsems[b])
        return carry

    lax.fori_loop(0, _NCHUNK // _NBUF, group, 0)

    for b in range(_NBUF):
        pltpu.make_async_copy(
            rows[b], out_hbm.at[pl.ds(0, _C)], osems[b]).wait()


_V = 100000
_PAD_RB = 2000


def _pad_body(i_ref, o_ref):
    o_ref[:, :_EMB] = i_ref[...]
    o_ref[:, _EMB:] = jnp.zeros((_PAD_RB, _DP - _EMB), jnp.float32)


_pad_tc = pl.pallas_call(
    _pad_body,
    grid=(_V // _PAD_RB,),
    in_specs=[pl.BlockSpec((_PAD_RB, _EMB), lambda i: (i, 0))],
    out_specs=pl.BlockSpec((_PAD_RB, _DP), lambda i: (i, 0)),
    out_shape=jax.ShapeDtypeStruct((_V, _DP), jnp.float32),
)


_SL_BB = 32


def _slice_body(i_ref, o_ref):
    o_ref[...] = i_ref[:, :, :_EMB]


_slice_tc = pl.pallas_call(
    _slice_body,
    grid=(_B // _SL_BB,),
    in_specs=[pl.BlockSpec((_SL_BB, _L, _DP), lambda i: (i, 0, 0))],
    out_specs=pl.BlockSpec((_SL_BB, _L, _EMB), lambda i: (i, 0, 0)),
    out_shape=jax.ShapeDtypeStruct((_B, _L, _EMB), jnp.float32),
)


def kernel(X, glove):
    idx = X.reshape(_NW * _NCHUNK, _C).astype(jnp.int32)
    glove_p = _pad_tc(glove)
    out = _gather(idx, glove_p)
    return _slice_tc(out.reshape(_B, _L, _DP))


# speedup vs baseline: 1.5075x; 1.5075x over previous
import functools

import jax
import jax.numpy as jnp
from jax import lax
from jax.experimental import pallas as pl
from jax.experimental.pallas import tpu as pltpu
from jax.experimental.pallas import tpu_sc as plsc

_B, _L, _EMB = 4096, 200, 100
_DP = 128
_NC, _NS = 2, 16
_NW = _NC * _NS
_BTOT = _B * _L
_BPW = _BTOT // _NW
_C = 128
_NCHUNK = _BPW // _C
_NBUF = 5

_mesh = plsc.VectorSubcoreMesh(core_axis_name="c", subcore_axis_name="s")


@functools.partial(
    pl.kernel,
    out_type=jax.ShapeDtypeStruct((_BTOT, _DP), jnp.float32),
    mesh=_mesh,
    scratch_types=[
        pltpu.VMEM((_NCHUNK, _C), jnp.int32),
        *[pltpu.VMEM((_C, _DP), jnp.float32) for _ in range(_NBUF)],
        *[pltpu.SemaphoreType.DMA for _ in range(2 * _NBUF)],
    ],
    compiler_params=pltpu.CompilerParams(use_tc_tiling_on_sc=True),
)
def _gather(idx_hbm, table_hbm, out_hbm, idx_v, *bufs_and_sems):
    rows = bufs_and_sems[:_NBUF]
    gsems = bufs_and_sems[_NBUF:2 * _NBUF]
    osems = bufs_and_sems[2 * _NBUF:]
    wid = lax.axis_index("s") * _NC + lax.axis_index("c")
    pltpu.sync_copy(idx_hbm.at[pl.ds(wid * _NCHUNK, _NCHUNK)], idx_v)
    base = wid * _BPW

    for b in range(_NBUF):
        pltpu.async_copy(table_hbm.at[idx_v.at[b]], rows[b], gsems[b])

    def group(go, carry):
        for b in range(_NBUF):
            g = go * _NBUF + b
            pltpu.make_async_copy(
                table_hbm.at[idx_v.at[0]], rows[b], gsems[b]).wait()
            pltpu.async_copy(
                rows[b], out_hbm.at[pl.ds((base + g * _C), _C)], osems[b])

            @pl.when(go < _NCHUNK // _NBUF - 1)
            def _():
                pltpu.make_async_copy(
                    rows[b], out_hbm.at[pl.ds(0, _C)], osems[b]).wait()
                pltpu.async_copy(
                    table_hbm.at[idx_v.at[g + _NBUF]], rows[b], gsems[b])
        return carry

    lax.fori_loop(0, _NCHUNK // _NBUF, group, 0)

    for b in range(_NBUF):
        pltpu.make_async_copy(
            rows[b], out_hbm.at[pl.ds(0, _C)], osems[b]).wait()


_V = 100000
_PAD_RB = 2000


def _pad_body(i_ref, o_ref):
    o_ref[:, :_EMB] = i_ref[...]
    o_ref[:, _EMB:] = jnp.zeros((_PAD_RB, _DP - _EMB), jnp.float32)


_pad_tc = pl.pallas_call(
    _pad_body,
    grid=(_V // _PAD_RB,),
    in_specs=[pl.BlockSpec((_PAD_RB, _EMB), lambda i: (i, 0))],
    out_specs=pl.BlockSpec((_PAD_RB, _DP), lambda i: (i, 0)),
    out_shape=jax.ShapeDtypeStruct((_V, _DP), jnp.float32),
)


def kernel(X, glove):
    idx = X.reshape(_NW * _NCHUNK, _C).astype(jnp.int32)
    glove_p = _pad_tc(glove)
    out = _gather(idx, glove_p)
    return out.reshape(_B, _L, _DP)[:, :, :_EMB]


# speedup vs baseline: 1.5111x; 1.0024x over previous
import functools

import jax
import jax.numpy as jnp
from jax import lax
from jax.experimental import pallas as pl
from jax.experimental.pallas import tpu as pltpu
from jax.experimental.pallas import tpu_sc as plsc

_B, _L, _EMB = 4096, 200, 100
_DP = 128
_NC, _NS = 2, 16
_NW = _NC * _NS
_BTOT = _B * _L
_BPW = _BTOT // _NW
_C = 128
_NCHUNK = _BPW // _C
_NBUF = 5

_mesh = plsc.VectorSubcoreMesh(core_axis_name="c", subcore_axis_name="s")


@functools.partial(
    pl.kernel,
    out_type=jax.ShapeDtypeStruct((_BTOT, _DP), jnp.float32),
    mesh=_mesh,
    scratch_types=[
        pltpu.VMEM((_NCHUNK, _C), jnp.int32),
        *[pltpu.VMEM((_C, _DP), jnp.float32) for _ in range(_NBUF)],
        *[pltpu.SemaphoreType.DMA for _ in range(2 * _NBUF)],
    ],
    compiler_params=pltpu.CompilerParams(use_tc_tiling_on_sc=True),
)
def _gather(idx_hbm, table_hbm, out_hbm, idx_v, *bufs_and_sems):
    rows = bufs_and_sems[:_NBUF]
    gsems = bufs_and_sems[_NBUF:2 * _NBUF]
    osems = bufs_and_sems[2 * _NBUF:]
    wid = lax.axis_index("s") * _NC + lax.axis_index("c")
    pltpu.sync_copy(idx_hbm.at[pl.ds(wid * _NCHUNK, _NCHUNK)], idx_v)
    base = wid * _BPW

    for b in range(_NBUF):
        pltpu.async_copy(table_hbm.at[idx_v.at[b]], rows[b], gsems[b])

    def group(go, carry):
        for b in range(_NBUF):
            g = go * _NBUF + b
            pltpu.make_async_copy(
                table_hbm.at[idx_v.at[0]], rows[b], gsems[b]).wait()
            pltpu.async_copy(
                rows[b], out_hbm.at[pl.ds((base + g * _C), _C)], osems[b])

            @pl.when(go < _NCHUNK // _NBUF - 1)
            def _():
                pltpu.make_async_copy(
                    rows[b], out_hbm.at[pl.ds(0, _C)], osems[b]).wait()
                pltpu.async_copy(
                    table_hbm.at[idx_v.at[g + _NBUF]], rows[b], gsems[b])
        return carry

    lax.fori_loop(0, _NCHUNK // _NBUF, group, 0)

    for b in range(_NBUF):
        pltpu.make_async_copy(
            rows[b], out_hbm.at[pl.ds(0, _C)], osems[b]).wait()


_V = 100000
_PAD_RB = 2000


def _pad_body(i_ref, o_ref):
    o_ref[:, :_EMB] = i_ref[...]


_pad_tc = pl.pallas_call(
    _pad_body,
    grid=(_V // _PAD_RB,),
    in_specs=[pl.BlockSpec((_PAD_RB, _EMB), lambda i: (i, 0))],
    out_specs=pl.BlockSpec((_PAD_RB, _DP), lambda i: (i, 0)),
    out_shape=jax.ShapeDtypeStruct((_V, _DP), jnp.float32),
)


def kernel(X, glove):
    idx = X.reshape(_NW * _NCHUNK, _C).astype(jnp.int32)
    glove_p = _pad_tc(glove)
    out = _gather(idx, glove_p)
    return out.reshape(_B, _L, _DP)[:, :, :_EMB]


# speedup vs baseline: 1.5424x; 1.0207x over previous
import functools

import jax
import jax.numpy as jnp
from jax import lax
from jax.experimental import pallas as pl
from jax.experimental.pallas import tpu as pltpu
from jax.experimental.pallas import tpu_sc as plsc

_B, _L, _EMB = 4096, 200, 100
_DP = 128
_NC, _NS = 2, 16
_NW = _NC * _NS
_BTOT = _B * _L
_BPW = _BTOT // _NW
_C = 128
_NCHUNK = _BPW // _C
_NBUF = 5

_mesh = plsc.VectorSubcoreMesh(core_axis_name="c", subcore_axis_name="s")


@functools.partial(
    pl.kernel,
    out_type=jax.ShapeDtypeStruct((_BTOT, _DP), jnp.float32),
    mesh=_mesh,
    scratch_types=[
        pltpu.VMEM((_NCHUNK, _C), jnp.int32),
        *[pltpu.VMEM((_C, _DP), jnp.float32) for _ in range(_NBUF)],
        *[pltpu.SemaphoreType.DMA for _ in range(2 * _NBUF)],
    ],
    compiler_params=pltpu.CompilerParams(use_tc_tiling_on_sc=True),
)
def _gather(idx_hbm, table_hbm, out_hbm, idx_v, *bufs_and_sems):
    rows = bufs_and_sems[:_NBUF]
    gsems = bufs_and_sems[_NBUF:2 * _NBUF]
    osems = bufs_and_sems[2 * _NBUF:]
    wid = lax.axis_index("s") * _NC + lax.axis_index("c")
    pltpu.sync_copy(idx_hbm.at[pl.ds(wid * _NCHUNK, _NCHUNK)], idx_v)
    base = wid * _BPW

    for b in range(_NBUF):
        pltpu.async_copy(table_hbm.at[idx_v.at[b]], rows[b], gsems[b])

    def group(go, carry):
        for b in range(_NBUF):
            g = go * _NBUF + b
            pltpu.make_async_copy(
                table_hbm.at[idx_v.at[0]], rows[b], gsems[b]).wait()
            pltpu.async_copy(
                rows[b], out_hbm.at[pl.ds((base + g * _C), _C)], osems[b])

            @pl.when(go < _NCHUNK // _NBUF - 1)
            def _():
                pltpu.make_async_copy(
                    rows[b], out_hbm.at[pl.ds(0, _C)], osems[b]).wait()
                pltpu.async_copy(
                    table_hbm.at[idx_v.at[g + _NBUF]], rows[b], gsems[b])
        return carry

    lax.fori_loop(0, _NCHUNK // _NBUF, group, 0)

    for b in range(_NBUF):
        pltpu.make_async_copy(
            rows[b], out_hbm.at[pl.ds(0, _C)], osems[b]).wait()


_V = 100000
_PAD_RB = 4000


def _pad_body(i_ref, o_ref):
    o_ref[:, :_EMB] = i_ref[...]


_pad_tc = pl.pallas_call(
    _pad_body,
    grid=(_V // _PAD_RB,),
    in_specs=[pl.BlockSpec((_PAD_RB, _EMB), lambda i: (i, 0))],
    out_specs=pl.BlockSpec((_PAD_RB, _DP), lambda i: (i, 0)),
    out_shape=jax.ShapeDtypeStruct((_V, _DP), jnp.float32),
)


def kernel(X, glove):
    idx = X.reshape(_NW * _NCHUNK, _C).astype(jnp.int32)
    glove_p = _pad_tc(glove)
    out = _gather(idx, glove_p)
    return out.reshape(_B, _L, _DP)[:, :, :_EMB]


# speedup vs baseline: 1.5505x; 1.0052x over previous
import functools

import jax
import jax.numpy as jnp
from jax import lax
from jax.experimental import pallas as pl
from jax.experimental.pallas import tpu as pltpu
from jax.experimental.pallas import tpu_sc as plsc

_B, _L, _EMB = 4096, 200, 100
_DP = 128
_NC, _NS = 2, 16
_NW = _NC * _NS
_BTOT = _B * _L
_BPW = _BTOT // _NW
_C = 128
_NCHUNK = _BPW // _C
_NBUF = 5

_mesh = plsc.VectorSubcoreMesh(core_axis_name="c", subcore_axis_name="s")


@functools.partial(
    pl.kernel,
    out_type=jax.ShapeDtypeStruct((_BTOT, _DP), jnp.float32),
    mesh=_mesh,
    scratch_types=[
        pltpu.VMEM((_NCHUNK, _C), jnp.int32),
        *[pltpu.VMEM((_C, _DP), jnp.float32) for _ in range(_NBUF)],
        *[pltpu.SemaphoreType.DMA for _ in range(2 * _NBUF)],
    ],
    compiler_params=pltpu.CompilerParams(use_tc_tiling_on_sc=True),
)
def _gather(idx_hbm, table_hbm, out_hbm, idx_v, *bufs_and_sems):
    rows = bufs_and_sems[:_NBUF]
    gsems = bufs_and_sems[_NBUF:2 * _NBUF]
    osems = bufs_and_sems[2 * _NBUF:]
    wid = lax.axis_index("s") * _NC + lax.axis_index("c")
    pltpu.sync_copy(idx_hbm.at[pl.ds(wid * _NCHUNK, _NCHUNK)], idx_v)
    base = wid * _BPW

    for b in range(_NBUF):
        pltpu.async_copy(table_hbm.at[idx_v.at[b]], rows[b], gsems[b])

    def group(go, carry):
        for b in range(_NBUF):
            g = go * _NBUF + b
            pltpu.make_async_copy(
                table_hbm.at[idx_v.at[0]], rows[b], gsems[b]).wait()
            pltpu.async_copy(
                rows[b], out_hbm.at[pl.ds((base + g * _C), _C)], osems[b])

            @pl.when(go < _NCHUNK // _NBUF - 1)
            def _():
                pltpu.make_async_copy(
                    rows[b], out_hbm.at[pl.ds(0, _C)], osems[b]).wait()
                pltpu.async_copy(
                    table_hbm.at[idx_v.at[g + _NBUF]], rows[b], gsems[b])
        return carry

    lax.fori_loop(0, _NCHUNK // _NBUF, group, 0)

    for b in range(_NBUF):
        pltpu.make_async_copy(
            rows[b], out_hbm.at[pl.ds(0, _C)], osems[b]).wait()


_V = 100000
_PAD_RB = 10000


def _pad_body(i_ref, o_ref):
    o_ref[:, :_EMB] = i_ref[...]


_pad_tc = pl.pallas_call(
    _pad_body,
    grid=(_V // _PAD_RB,),
    in_specs=[pl.BlockSpec((_PAD_RB, _EMB), lambda i: (i, 0))],
    out_specs=pl.BlockSpec((_PAD_RB, _DP), lambda i: (i, 0)),
    out_shape=jax.ShapeDtypeStruct((_V, _DP), jnp.float32),
)


def kernel(X, glove):
    idx = X.reshape(_NW * _NCHUNK, _C).astype(jnp.int32)
    glove_p = _pad_tc(glove)
    out = _gather(idx, glove_p)
    return out.reshape(_B, _L, _DP)[:, :, :_EMB]


# speedup vs baseline: 1.5536x; 1.0020x over previous
import functools

import jax
import jax.numpy as jnp
from jax import lax
from jax.experimental import pallas as pl
from jax.experimental.pallas import tpu as pltpu
from jax.experimental.pallas import tpu_sc as plsc

_B, _L, _EMB = 4096, 200, 100
_DP = 128
_NC, _NS = 2, 16
_NW = _NC * _NS
_BTOT = _B * _L
_BPW = _BTOT // _NW
_C = 128
_NCHUNK = _BPW // _C
_NBUF = 5

_mesh = plsc.VectorSubcoreMesh(core_axis_name="c", subcore_axis_name="s")


@functools.partial(
    pl.kernel,
    out_type=jax.ShapeDtypeStruct((_BTOT, _DP), jnp.float32),
    mesh=_mesh,
    scratch_types=[
        pltpu.VMEM((_NCHUNK, _C), jnp.int32),
        *[pltpu.VMEM((_C, _DP), jnp.float32) for _ in range(_NBUF)],
        *[pltpu.SemaphoreType.DMA for _ in range(2 * _NBUF)],
    ],
    compiler_params=pltpu.CompilerParams(use_tc_tiling_on_sc=True),
)
def _gather(idx_hbm, table_hbm, out_hbm, idx_v, *bufs_and_sems):
    rows = bufs_and_sems[:_NBUF]
    gsems = bufs_and_sems[_NBUF:2 * _NBUF]
    osems = bufs_and_sems[2 * _NBUF:]
    wid = lax.axis_index("s") * _NC + lax.axis_index("c")
    pltpu.sync_copy(idx_hbm.at[pl.ds(wid * _NCHUNK, _NCHUNK)], idx_v)
    base = wid * _BPW

    for b in range(_NBUF):
        pltpu.async_copy(table_hbm.at[idx_v.at[b]], rows[b], gsems[b])

    def group(go, carry):
        for b in range(_NBUF):
            g = go * _NBUF + b
            pltpu.make_async_copy(
                table_hbm.at[idx_v.at[0]], rows[b], gsems[b]).wait()
            pltpu.async_copy(
                rows[b], out_hbm.at[pl.ds((base + g * _C), _C)], osems[b])

            @pl.when(go < _NCHUNK // _NBUF - 1)
            def _():
                pltpu.make_async_copy(
                    rows[b], out_hbm.at[pl.ds(0, _C)], osems[b]).wait()
                pltpu.async_copy(
                    table_hbm.at[idx_v.at[g + _NBUF]], rows[b], gsems[b])
        return carry

    lax.fori_loop(0, _NCHUNK // _NBUF, group, 0)

    for b in range(_NBUF):
        pltpu.make_async_copy(
            rows[b], out_hbm.at[pl.ds(0, _C)], osems[b]).wait()


_V = 100000
_PAD_RB = 25000


def _pad_body(i_ref, o_ref):
    o_ref[:, :_EMB] = i_ref[...]


_pad_tc = pl.pallas_call(
    _pad_body,
    grid=(_V // _PAD_RB,),
    in_specs=[pl.BlockSpec((_PAD_RB, _EMB), lambda i: (i, 0))],
    out_specs=pl.BlockSpec((_PAD_RB, _DP), lambda i: (i, 0)),
    out_shape=jax.ShapeDtypeStruct((_V, _DP), jnp.float32),
)


def kernel(X, glove):
    idx = X.reshape(_NW * _NCHUNK, _C).astype(jnp.int32)
    glove_p = _pad_tc(glove)
    out = _gather(idx, glove_p)
    return out.reshape(_B, _L, _DP)[:, :, :_EMB]
